# baseline (speedup 1.0000x reference)
import functools

import jax
import jax.numpy as jnp
from jax import lax
from jax.experimental import pallas as pl
from jax.experimental.pallas import tpu as pltpu
from jax.experimental.pallas import tpu_sc as plsc

N_ITEM = 1000000
N_DIM = 64
BATCH = 16384

NC = 2
NS = 16
L = 16
NW = NC * NS
B_PER_W = BATCH // NW
CHUNK = 128
N_CHUNKS = B_PER_W // CHUNK


def _sc_body(word_hbm, ctx_hbm, wtab_hbm, ctab_hbm, out_hbm,
             widx, cidx, wrows, crows, out_v, sems):
    wid = lax.axis_index("s") * NC + lax.axis_index("c")

    pltpu.sync_copy(word_hbm.at[pl.ds(wid * N_CHUNKS, N_CHUNKS)], widx)
    pltpu.sync_copy(ctx_hbm.at[pl.ds(wid * N_CHUNKS, N_CHUNKS)], cidx)

    copies = []
    for c in range(N_CHUNKS):
        copies.append(pltpu.async_copy(
            wtab_hbm.at[widx.at[c]], wrows.at[pl.ds(c * CHUNK, CHUNK)],
            sems.at[2 * c]))
        copies.append(pltpu.async_copy(
            ctab_hbm.at[cidx.at[c]], crows.at[pl.ds(c * CHUNK, CHUNK)],
            sems.at[2 * c + 1]))

    for c in range(N_CHUNKS):
        copies[2 * c].wait()
        copies[2 * c + 1].wait()

        def body(g, _, c=c):
            base = c * CHUNK + g * L
            ridx = jnp.arange(L, dtype=jnp.int32) + base
            acc = jnp.zeros((L,), jnp.float32)
            for j in range(N_DIM):
                cj = jnp.full((L,), j, jnp.int32)
                w = plsc.load_gather(wrows, [ridx, cj])
                x = plsc.load_gather(crows, [ridx, cj])
                acc = acc + w * x
            sig = 1.0 / (1.0 + jnp.exp(-acc))
            out_v[pl.ds(base, L)] = sig
            return 0

        lax.fori_loop(0, CHUNK // L, body, 0)

    pltpu.sync_copy(out_v, out_hbm.at[pl.ds(wid * B_PER_W, B_PER_W)])


@jax.jit
def _skipgram_sc(word2d, ctx2d, wtab, ctab):
    mesh = plsc.VectorSubcoreMesh(core_axis_name="c", subcore_axis_name="s",
                                  num_cores=NC, num_subcores=NS)
    return pl.kernel(
        _sc_body,
        out_type=jax.ShapeDtypeStruct((BATCH,), jnp.float32),
        mesh=mesh,
        compiler_params=pltpu.CompilerParams(needs_layout_passes=False,
                                             use_tc_tiling_on_sc=False),
        scratch_types=[
            pltpu.VMEM((N_CHUNKS, CHUNK), jnp.int32),
            pltpu.VMEM((N_CHUNKS, CHUNK), jnp.int32),
            pltpu.VMEM((B_PER_W, N_DIM), jnp.float32),
            pltpu.VMEM((B_PER_W, N_DIM), jnp.float32),
            pltpu.VMEM((B_PER_W,), jnp.float32),
            pltpu.SemaphoreType.DMA((2 * N_CHUNKS,)),
        ],
    )(word2d, ctx2d, wtab, ctab)


def kernel(word, context, word_embeddings, context_embeddings):
    word2d = word.astype(jnp.int32).reshape(NW * N_CHUNKS, CHUNK)
    ctx2d = context.astype(jnp.int32).reshape(NW * N_CHUNKS, CHUNK)
    return _skipgram_sc(word2d, ctx2d, word_embeddings, context_embeddings)
